# baseline (speedup 1.0000x reference)
import functools

import jax
import jax.numpy as jnp
from jax import lax
from jax.experimental import pallas as pl
from jax.experimental.pallas import tpu as pltpu
from jax.experimental.pallas import tpu_sc as plsc

T = 64
D = 768
E = 64
F = 1024

_L = 16
_NW = 32
_TPW = T // _NW


def _take(v, idx):
    return v.at[idx].get(mode="promise_in_bounds")


def _route_sc_body(logits_hbm, gates_hbm, row_v, out_v):
    wid = lax.axis_index("s") * 2 + lax.axis_index("c")
    iota16 = lax.iota(jnp.int32, _L)
    for j in range(_TPW):
        t = wid * _TPW + j
        pltpu.sync_copy(logits_hbm.at[t], row_v)
        cand_k = jnp.full((_L,), -3.0e38, jnp.float32)
        cand_i = jnp.full((_L,), E, jnp.int32)
        for c in range(E // _L):
            v = row_v[pl.ds(c * _L, _L)]
            gi = iota16 + c * _L
            sk, si = plsc.sort_key_val(v, gi, descending=True)
            sel = (iota16 >= 2 * c) & (iota16 < 2 * c + 2)
            sh = jnp.maximum(iota16 - 2 * c, 0)
            cand_k = jnp.where(sel, _take(sk, sh), cand_k)
            cand_i = jnp.where(sel, _take(si, sh), cand_i)
        fk, fi = plsc.sort_key_val(cand_k, cand_i, descending=True)
        zero = jnp.zeros((_L,), jnp.int32)
        m1 = _take(fk, zero)
        m2 = _take(fk, zero + 1)
        i1 = _take(fi, zero)
        i2 = _take(fi, zero + 1)
        d = jnp.exp(m2 - m1)
        w1 = 1.0 / (1.0 + d)
        w2 = d / (1.0 + d)
        for c in range(E // _L):
            gi = iota16 + c * _L
            out_v[pl.ds(c * _L, _L)] = jnp.where(
                gi == i1, w1, jnp.where(gi == i2, w2, 0.0))
        pltpu.sync_copy(out_v, gates_hbm.at[t])


def _route_sc(router_logits):
    mesh = plsc.VectorSubcoreMesh(core_axis_name="c", subcore_axis_name="s")
    return functools.partial(
        pl.kernel, mesh=mesh,
        out_type=jax.ShapeDtypeStruct((T, E), jnp.float32),
        scratch_types=[
            pltpu.VMEM((E,), jnp.float32),
            pltpu.VMEM((E,), jnp.float32),
        ],
        compiler_params=pltpu.CompilerParams(needs_layout_passes=False),
    )(_route_sc_body)(router_logits)


def _moe_body(h_ref, gin_ref, scale_ref, wg_hbm, wu_hbm, wd_hbm, out_ref,
              gates_ref, alist_v, cnt_v, alist_s, cnt_s,
              wg_buf, wu_buf, wd_buf, sems, lsem):
    i = pl.program_id(0)

    def start(eid, slot):
        pltpu.make_async_copy(wg_hbm.at[eid], wg_buf.at[slot],
                              sems.at[slot, 0]).start()
        pltpu.make_async_copy(wu_hbm.at[eid], wu_buf.at[slot],
                              sems.at[slot, 1]).start()
        pltpu.make_async_copy(wd_hbm.at[eid], wd_buf.at[slot],
                              sems.at[slot, 2]).start()

    @pl.when(i == 0)
    def _():
        start(0, 0)
        out_ref[...] = jnp.zeros_like(out_ref)

        g = gin_ref[...] * scale_ref[...]
        gates_ref[...] = g

        cnt = jnp.sum((gin_ref[...] > 0.0).astype(jnp.int32), axis=0)
        iota_e = lax.iota(jnp.int32, E)
        act = (cnt > 0) & (iota_e > 0)
        rowi = lax.broadcasted_iota(jnp.int32, (E, E), 0)
        coli = lax.broadcasted_iota(jnp.int32, (E, E), 1)
        before = (coli < rowi) & act[None, :]
        rank = jnp.sum(before.astype(jnp.int32), axis=1) + 1
        hits = act[None, :] & (rank[None, :] == rowi)
        alist_v[...] = jnp.sum(jnp.where(hits, coli, 0), axis=1).reshape(1, E)
        cnt_v[...] = (jnp.sum(act.astype(jnp.int32)) + 1).reshape(1, 1)
        pltpu.make_async_copy(alist_v, alist_s, lsem.at[0]).start()
        pltpu.make_async_copy(cnt_v, cnt_s, lsem.at[1]).start()
        pltpu.make_async_copy(alist_v, alist_s, lsem.at[0]).wait()
        pltpu.make_async_copy(cnt_v, cnt_s, lsem.at[1]).wait()

    n = cnt_s[0, 0]

    @pl.when(i + 1 < n)
    def _():
        start(alist_s[0, i + 1], jax.lax.rem(i + 1, 2))

    @pl.when(i < n)
    def _():
        slot = jax.lax.rem(i, 2)
        eid = alist_s[0, i]
        pltpu.make_async_copy(wg_hbm.at[eid], wg_buf.at[slot],
                              sems.at[slot, 0]).wait()
        pltpu.make_async_copy(wu_hbm.at[eid], wu_buf.at[slot],
                              sems.at[slot, 1]).wait()
        pltpu.make_async_copy(wd_hbm.at[eid], wd_buf.at[slot],
                              sems.at[slot, 2]).wait()
        h = h_ref[...]
        g = jax.nn.gelu(
            jnp.dot(h, wg_buf[slot], preferred_element_type=jnp.float32))
        u = jnp.dot(h, wu_buf[slot], preferred_element_type=jnp.float32)
        y = jnp.dot(g * u, wd_buf[slot], preferred_element_type=jnp.float32)
        lane = lax.broadcasted_iota(jnp.int32, (T, E), 1)
        gcol = jnp.sum(jnp.where(lane == eid, gates_ref[...], 0.0),
                       axis=1, keepdims=True)
        out_ref[...] += y * gcol


def kernel(hidden_states, router_logits, w_gate, w_up, w_down,
           per_expert_scale):
    gates_raw = _route_sc(router_logits)
    scale2d = per_expert_scale.reshape(1, E)
    return pl.pallas_call(
        _moe_body,
        grid=(E,),
        in_specs=[
            pl.BlockSpec((T, D), lambda i: (0, 0)),
            pl.BlockSpec((T, E), lambda i: (0, 0)),
            pl.BlockSpec((1, E), lambda i: (0, 0)),
            pl.BlockSpec(memory_space=pl.ANY),
            pl.BlockSpec(memory_space=pl.ANY),
            pl.BlockSpec(memory_space=pl.ANY),
        ],
        out_specs=pl.BlockSpec((T, D), lambda i: (0, 0)),
        out_shape=jax.ShapeDtypeStruct((T, D), jnp.float32),
        scratch_shapes=[
            pltpu.VMEM((T, E), jnp.float32),
            pltpu.VMEM((1, E), jnp.int32),
            pltpu.VMEM((1, 1), jnp.int32),
            pltpu.SMEM((1, E), jnp.int32),
            pltpu.SMEM((1, 1), jnp.int32),
            pltpu.VMEM((2, D, F), jnp.float32),
            pltpu.VMEM((2, D, F), jnp.float32),
            pltpu.VMEM((2, F, D), jnp.float32),
            pltpu.SemaphoreType.DMA((2, 3)),
            pltpu.SemaphoreType.DMA((2,)),
        ],
        compiler_params=pltpu.CompilerParams(
            dimension_semantics=("arbitrary",)),
    )(hidden_states, gates_raw, scale2d, w_gate, w_up, w_down)


# speedup vs baseline: 1.1384x; 1.1384x over previous
import jax
import jax.numpy as jnp
from jax.experimental import pallas as pl
from jax.experimental.pallas import tpu as pltpu

T = 64
D = 768
E = 64
F = 1024


def _moe_body(h_ref, logits_ref, scale_ref, wg_hbm, wu_hbm, wd_hbm, out_ref,
              gates_ref, alist_v, cnt_v, alist_s, cnt_s,
              wg_buf, wu_buf, wd_buf, sems, lsem):
    i = pl.program_id(0)

    def start(eid, slot):
        pltpu.make_async_copy(wg_hbm.at[eid], wg_buf.at[slot],
                              sems.at[slot, 0]).start()
        pltpu.make_async_copy(wu_hbm.at[eid], wu_buf.at[slot],
                              sems.at[slot, 1]).start()
        pltpu.make_async_copy(wd_hbm.at[eid], wd_buf.at[slot],
                              sems.at[slot, 2]).start()

    @pl.when(i == 0)
    def _():
        start(0, 0)
        start(1, 1)
        out_ref[...] = jnp.zeros_like(out_ref)

        logits = logits_ref[...]
        lane = jax.lax.broadcasted_iota(jnp.int32, (T, E), 1)
        a1 = jnp.argmax(logits, axis=1)
        oh1 = lane == a1[:, None]
        masked = jnp.where(oh1, -jnp.inf, logits)
        a2 = jnp.argmax(masked, axis=1)
        oh2 = lane == a2[:, None]
        probs = jax.nn.softmax(logits, axis=1)
        sel = jnp.where(oh1 | oh2, probs, 0.0)
        renorm = jnp.sum(sel, axis=1, keepdims=True)
        renorm = jnp.where(renorm > 0.0, renorm, 1.0)
        gates_ref[...] = sel / renorm * scale_ref[...]

        cnt = jnp.sum((oh1 | oh2).astype(jnp.int32), axis=0)
        iota_e = jax.lax.iota(jnp.int32, E)
        act = (cnt > 0) & (iota_e > 1)
        rowi = jax.lax.broadcasted_iota(jnp.int32, (E, E), 0)
        coli = jax.lax.broadcasted_iota(jnp.int32, (E, E), 1)
        before = (coli < rowi) & act[None, :]
        rank = jnp.sum(before.astype(jnp.int32), axis=1) + 2
        hits = act[None, :] & (rank[None, :] == rowi)
        alist = jnp.sum(jnp.where(hits, coli, 0), axis=1)
        alist = jnp.where(iota_e == 1, 1, alist)
        alist_v[...] = alist.reshape(1, E)
        cnt_v[...] = (jnp.sum(act.astype(jnp.int32)) + 2).reshape(1, 1)
        pltpu.make_async_copy(alist_v, alist_s, lsem.at[0]).start()
        pltpu.make_async_copy(cnt_v, cnt_s, lsem.at[1]).start()
        pltpu.make_async_copy(alist_v, alist_s, lsem.at[0]).wait()
        pltpu.make_async_copy(cnt_v, cnt_s, lsem.at[1]).wait()

    n = cnt_s[0, 0]

    @pl.when((i >= 1) & (i + 1 < n))
    def _():
        start(alist_s[0, i + 1], jax.lax.rem(i + 1, 2))

    @pl.when(i < n)
    def _():
        slot = jax.lax.rem(i, 2)
        eid = alist_s[0, i]
        pltpu.make_async_copy(wg_hbm.at[eid], wg_buf.at[slot],
                              sems.at[slot, 0]).wait()
        pltpu.make_async_copy(wu_hbm.at[eid], wu_buf.at[slot],
                              sems.at[slot, 1]).wait()
        pltpu.make_async_copy(wd_hbm.at[eid], wd_buf.at[slot],
                              sems.at[slot, 2]).wait()
        h = h_ref[...]
        g = jax.nn.gelu(
            jnp.dot(h, wg_buf[slot], preferred_element_type=jnp.float32))
        u = jnp.dot(h, wu_buf[slot], preferred_element_type=jnp.float32)
        y = jnp.dot(g * u, wd_buf[slot], preferred_element_type=jnp.float32)
        lane = jax.lax.broadcasted_iota(jnp.int32, (T, E), 1)
        gcol = jnp.sum(jnp.where(lane == eid, gates_ref[...], 0.0),
                       axis=1, keepdims=True)
        out_ref[...] += y * gcol


def kernel(hidden_states, router_logits, w_gate, w_up, w_down,
           per_expert_scale):
    scale2d = per_expert_scale.reshape(1, E)
    return pl.pallas_call(
        _moe_body,
        grid=(E,),
        in_specs=[
            pl.BlockSpec((T, D), lambda i: (0, 0)),
            pl.BlockSpec((T, E), lambda i: (0, 0)),
            pl.BlockSpec((1, E), lambda i: (0, 0)),
            pl.BlockSpec(memory_space=pl.ANY),
            pl.BlockSpec(memory_space=pl.ANY),
            pl.BlockSpec(memory_space=pl.ANY),
        ],
        out_specs=pl.BlockSpec((T, D), lambda i: (0, 0)),
        out_shape=jax.ShapeDtypeStruct((T, D), jnp.float32),
        scratch_shapes=[
            pltpu.VMEM((T, E), jnp.float32),
            pltpu.VMEM((1, E), jnp.int32),
            pltpu.VMEM((1, 1), jnp.int32),
            pltpu.SMEM((1, E), jnp.int32),
            pltpu.SMEM((1, 1), jnp.int32),
            pltpu.VMEM((2, D, F), jnp.float32),
            pltpu.VMEM((2, D, F), jnp.float32),
            pltpu.VMEM((2, F, D), jnp.float32),
            pltpu.SemaphoreType.DMA((2, 3)),
            pltpu.SemaphoreType.DMA((2,)),
        ],
        compiler_params=pltpu.CompilerParams(
            dimension_semantics=("arbitrary",)),
    )(hidden_states, router_logits, scale2d, w_gate, w_up, w_down)
